# Initial kernel scaffold; baseline (speedup 1.0000x reference)
import jax
import jax.numpy as jnp
from jax.experimental import pallas as pl


def kernel(x, edge_index, W_self1, W_neigh1, b1, W_self2, W_neigh2, b2):
    raise NotImplementedError("write your pallas kernel here")


# speedup vs baseline: 4.7781x; 4.7781x over previous
import functools

import jax
import jax.numpy as jnp
from jax import lax
from jax.experimental import pallas as pl
from jax.experimental.pallas import tpu as pltpu
from jax.experimental.pallas import tpu_sc as plsc

N = 10000
D = 128
DP = 144
E = 320000
NC = 2
NS = 16
NW = NC * NS
EPW = E // NW
B = 80
NB = EPW // B
RPT = N // NS


def _make_agg_body(w):
    def body(tab_hbm, src_hbm, dst_hbm, zz_hbm, agg_hbm,
             srcv, dstv, rows, acc, sem):
        c = lax.axis_index("c")
        s = lax.axis_index("s")
        wid = s * NC + c
        pltpu.sync_copy(zz_hbm.at[s], acc.at[pl.ds(s * RPT, RPT)])
        plsc.subcore_barrier()

        ebase = wid * EPW

        def step(i, carry):
            base = ebase + i * B
            pltpu.sync_copy(src_hbm.at[pl.ds(base, B)], srcv)
            pltpu.sync_copy(dst_hbm.at[pl.ds(base, B)], dstv)
            pltpu.async_copy(tab_hbm.at[srcv], rows, sem).wait()
            pltpu.sync_copy(rows, acc.at[dstv], add=True)
            return carry

        lax.fori_loop(0, NB, step, 0)
        plsc.subcore_barrier()
        pltpu.sync_copy(acc.at[pl.ds(s * RPT, RPT)], agg_hbm.at[c * NS + s])

    return body


def _sc_agg(table, src, dst, zz, w):
    return pl.kernel(
        _make_agg_body(w),
        out_type=jax.ShapeDtypeStruct((NW, RPT, w), jnp.float32),
        mesh=plsc.VectorSubcoreMesh(core_axis_name="c", subcore_axis_name="s"),
        compiler_params=pltpu.CompilerParams(use_tc_tiling_on_sc=False),
        scratch_types=(
            pltpu.VMEM((B,), jnp.int32),
            pltpu.VMEM((B,), jnp.int32),
            pltpu.VMEM((B, w), jnp.float32),
            pltpu.VMEM_SHARED((N, w), jnp.float32),
            pltpu.SemaphoreType.DMA,
        ),
    )(table, src, dst, zz)


BLK = 1000
NBLK = N // BLK


def _layer1_body(x_ref, a0_ref, a1_ref, ws_ref, wn_ref, b_ref, o_ref):
    a = a0_ref[...] + a1_ref[...]
    rdeg = 1.0 / jnp.maximum(a[:, D:D + 1], 1.0)
    neigh = a[:, :D] * rdeg
    h = (jnp.dot(x_ref[...], ws_ref[...], preferred_element_type=jnp.float32)
         + jnp.dot(neigh, wn_ref[...], preferred_element_type=jnp.float32)
         + b_ref[...])
    o_ref[...] = jnp.maximum(h, 0.0)


def _layer2_body(x_ref, a0_ref, a1_ref, d0_ref, d1_ref, ws_ref, wn_ref,
                 b_ref, o_ref):
    d = d0_ref[:, D:D + 1] + d1_ref[:, D:D + 1]
    rdeg = 1.0 / jnp.maximum(d, 1.0)
    neigh = (a0_ref[...] + a1_ref[...]) * rdeg
    h = (jnp.dot(x_ref[...], ws_ref[...], preferred_element_type=jnp.float32)
         + jnp.dot(neigh, wn_ref[...], preferred_element_type=jnp.float32)
         + b_ref[...])
    o_ref[...] = h


def _tc_layer1(x, agg, ws_t, wn_t, b):
    return pl.pallas_call(
        _layer1_body,
        grid=(NBLK,),
        in_specs=[
            pl.BlockSpec((BLK, D), lambda i: (i, 0)),
            pl.BlockSpec((BLK, DP), lambda i: (i, 0)),
            pl.BlockSpec((BLK, DP), lambda i: (i + NBLK, 0)),
            pl.BlockSpec((D, D), lambda i: (0, 0)),
            pl.BlockSpec((D, D), lambda i: (0, 0)),
            pl.BlockSpec((1, D), lambda i: (0, 0)),
        ],
        out_specs=pl.BlockSpec((BLK, D), lambda i: (i, 0)),
        out_shape=jax.ShapeDtypeStruct((N, D), jnp.float32),
    )(x, agg, agg, ws_t, wn_t, b)


def _tc_layer2(x, agg, degp, ws_t, wn_t, b):
    return pl.pallas_call(
        _layer2_body,
        grid=(NBLK,),
        in_specs=[
            pl.BlockSpec((BLK, D), lambda i: (i, 0)),
            pl.BlockSpec((BLK, D), lambda i: (i, 0)),
            pl.BlockSpec((BLK, D), lambda i: (i + NBLK, 0)),
            pl.BlockSpec((BLK, DP), lambda i: (i, 0)),
            pl.BlockSpec((BLK, DP), lambda i: (i + NBLK, 0)),
            pl.BlockSpec((D, D), lambda i: (0, 0)),
            pl.BlockSpec((D, D), lambda i: (0, 0)),
            pl.BlockSpec((1, D), lambda i: (0, 0)),
        ],
        out_specs=pl.BlockSpec((BLK, D), lambda i: (i, 0)),
        out_shape=jax.ShapeDtypeStruct((N, D), jnp.float32),
    )(x, agg, agg, degp, degp, ws_t, wn_t, b)


def kernel(x, edge_index, W_self1, W_neigh1, b1, W_self2, W_neigh2, b2):
    src = edge_index[0].astype(jnp.int32)
    dst = edge_index[1].astype(jnp.int32)
    xa = jnp.concatenate(
        [x, jnp.ones((N, 1), jnp.float32), jnp.zeros((N, DP - D - 1),
                                                     jnp.float32)], axis=1)
    zz1 = jnp.zeros((NS, RPT, DP), jnp.float32)
    zz2 = jnp.zeros((NS, RPT, D), jnp.float32)
    agg1 = _sc_agg(xa, src, dst, zz1, DP).reshape(NC * N, DP)
    h1 = _tc_layer1(x, agg1, W_self1.T, W_neigh1.T, b1[None, :])
    agg2 = _sc_agg(h1, src, dst, zz2, D).reshape(NC * N, D)
    return _tc_layer2(h1, agg2, agg1, W_self2.T, W_neigh2.T, b2[None, :])


# speedup vs baseline: 8.5998x; 1.7998x over previous
import functools

import jax
import jax.numpy as jnp
from jax import lax
from jax.experimental import pallas as pl
from jax.experimental.pallas import tpu as pltpu
from jax.experimental.pallas import tpu_sc as plsc

N = 10000
D = 128
DP = 144
E = 320000
NC = 2
NS = 16
NW = NC * NS
EPW = E // NW
B = 125
NB = EPW // B
RPT = N // NS


def _make_agg_body(w):
    def body(tab_hbm, src_hbm, dst_hbm, zz_hbm, agg_hbm,
             sb0, db0, sb1, db1, rows0, rows1, acc,
             semi0, semi1, semg0, semg1):
        c = lax.axis_index("c")
        s = lax.axis_index("s")
        wid = s * NC + c
        pltpu.sync_copy(zz_hbm.at[s], acc.at[pl.ds(s * RPT, RPT)])
        plsc.subcore_barrier()

        ibase = wid * NB

        def iload(i, sb, db, sem):
            pltpu.async_copy(src_hbm.at[ibase + i], sb, sem)
            pltpu.async_copy(dst_hbm.at[ibase + i], db, sem)

        def iwait(sb, db, sem):
            pltpu.make_async_copy(src_hbm.at[0], sb, sem).wait()
            pltpu.make_async_copy(dst_hbm.at[0], db, sem).wait()

        def gather(sb, rows, sem):
            pltpu.async_copy(tab_hbm.at[sb], rows, sem)

        def gwait(rows, sem):
            pltpu.make_async_copy(tab_hbm.at[sb0], rows, sem).wait()

        def scat(db, rows):
            pltpu.sync_copy(rows, acc.at[db], add=True)

        iload(0, sb0, db0, semi0)
        iload(1, sb1, db1, semi1)
        iwait(sb0, db0, semi0)
        gather(sb0, rows0, semg0)
        iwait(sb1, db1, semi1)
        gather(sb1, rows1, semg1)

        def step(j, carry):
            i0 = 2 * j
            gwait(rows0, semg0)
            scat(db0, rows0)
            iload(i0 + 2, sb0, db0, semi0)
            gwait(rows1, semg1)
            scat(db1, rows1)
            iload(i0 + 3, sb1, db1, semi1)
            iwait(sb0, db0, semi0)
            gather(sb0, rows0, semg0)
            iwait(sb1, db1, semi1)
            gather(sb1, rows1, semg1)
            return carry

        lax.fori_loop(0, NB // 2 - 1, step, 0)
        gwait(rows0, semg0)
        scat(db0, rows0)
        gwait(rows1, semg1)
        scat(db1, rows1)
        plsc.subcore_barrier()
        pltpu.sync_copy(acc.at[pl.ds(s * RPT, RPT)], agg_hbm.at[c * NS + s])

    return body


def _sc_agg(table, src, dst, zz, w):
    return pl.kernel(
        _make_agg_body(w),
        out_type=jax.ShapeDtypeStruct((NW, RPT, w), jnp.float32),
        mesh=plsc.VectorSubcoreMesh(core_axis_name="c", subcore_axis_name="s"),
        compiler_params=pltpu.CompilerParams(use_tc_tiling_on_sc=False),
        scratch_types=(
            pltpu.VMEM((B,), jnp.int32),
            pltpu.VMEM((B,), jnp.int32),
            pltpu.VMEM((B,), jnp.int32),
            pltpu.VMEM((B,), jnp.int32),
            pltpu.VMEM((B, w), jnp.float32),
            pltpu.VMEM((B, w), jnp.float32),
            pltpu.VMEM_SHARED((N, w), jnp.float32),
            pltpu.SemaphoreType.DMA,
            pltpu.SemaphoreType.DMA,
            pltpu.SemaphoreType.DMA,
            pltpu.SemaphoreType.DMA,
        ),
    )(table, src, dst, zz)


BLK = 1000
NBLK = N // BLK


def _layer1_body(x_ref, a0_ref, a1_ref, ws_ref, wn_ref, b_ref, o_ref):
    a = a0_ref[...] + a1_ref[...]
    rdeg = 1.0 / jnp.maximum(a[:, D:D + 1], 1.0)
    neigh = a[:, :D] * rdeg
    h = (jnp.dot(x_ref[...], ws_ref[...], preferred_element_type=jnp.float32)
         + jnp.dot(neigh, wn_ref[...], preferred_element_type=jnp.float32)
         + b_ref[...])
    o_ref[...] = jnp.maximum(h, 0.0)


def _layer2_body(x_ref, a0_ref, a1_ref, d0_ref, d1_ref, ws_ref, wn_ref,
                 b_ref, o_ref):
    d = d0_ref[:, D:D + 1] + d1_ref[:, D:D + 1]
    rdeg = 1.0 / jnp.maximum(d, 1.0)
    neigh = (a0_ref[...] + a1_ref[...]) * rdeg
    h = (jnp.dot(x_ref[...], ws_ref[...], preferred_element_type=jnp.float32)
         + jnp.dot(neigh, wn_ref[...], preferred_element_type=jnp.float32)
         + b_ref[...])
    o_ref[...] = h


def _tc_layer1(x, agg, ws_t, wn_t, b):
    return pl.pallas_call(
        _layer1_body,
        grid=(NBLK,),
        in_specs=[
            pl.BlockSpec((BLK, D), lambda i: (i, 0)),
            pl.BlockSpec((BLK, DP), lambda i: (i, 0)),
            pl.BlockSpec((BLK, DP), lambda i: (i + NBLK, 0)),
            pl.BlockSpec((D, D), lambda i: (0, 0)),
            pl.BlockSpec((D, D), lambda i: (0, 0)),
            pl.BlockSpec((1, D), lambda i: (0, 0)),
        ],
        out_specs=pl.BlockSpec((BLK, D), lambda i: (i, 0)),
        out_shape=jax.ShapeDtypeStruct((N, D), jnp.float32),
    )(x, agg, agg, ws_t, wn_t, b)


def _tc_layer2(x, agg, degp, ws_t, wn_t, b):
    return pl.pallas_call(
        _layer2_body,
        grid=(NBLK,),
        in_specs=[
            pl.BlockSpec((BLK, D), lambda i: (i, 0)),
            pl.BlockSpec((BLK, D), lambda i: (i, 0)),
            pl.BlockSpec((BLK, D), lambda i: (i + NBLK, 0)),
            pl.BlockSpec((BLK, DP), lambda i: (i, 0)),
            pl.BlockSpec((BLK, DP), lambda i: (i + NBLK, 0)),
            pl.BlockSpec((D, D), lambda i: (0, 0)),
            pl.BlockSpec((D, D), lambda i: (0, 0)),
            pl.BlockSpec((1, D), lambda i: (0, 0)),
        ],
        out_specs=pl.BlockSpec((BLK, D), lambda i: (i, 0)),
        out_shape=jax.ShapeDtypeStruct((N, D), jnp.float32),
    )(x, agg, agg, degp, degp, ws_t, wn_t, b)


def kernel(x, edge_index, W_self1, W_neigh1, b1, W_self2, W_neigh2, b2):
    src = edge_index[0].astype(jnp.int32).reshape(NW * NB, B)
    dst = edge_index[1].astype(jnp.int32).reshape(NW * NB, B)
    xa = jnp.concatenate(
        [x, jnp.ones((N, 1), jnp.float32), jnp.zeros((N, DP - D - 1),
                                                     jnp.float32)], axis=1)
    zz1 = jnp.zeros((NS, RPT, DP), jnp.float32)
    zz2 = jnp.zeros((NS, RPT, D), jnp.float32)
    agg1 = _sc_agg(xa, src, dst, zz1, DP).reshape(NC * N, DP)
    h1 = _tc_layer1(x, agg1, W_self1.T, W_neigh1.T, b1[None, :])
    agg2 = _sc_agg(h1, src, dst, zz2, D).reshape(NC * N, D)
    return _tc_layer2(h1, agg2, agg1, W_self2.T, W_neigh2.T, b2[None, :])


# speedup vs baseline: 9.7721x; 1.1363x over previous
import functools

import jax
import jax.numpy as jnp
from jax import lax
from jax.experimental import pallas as pl
from jax.experimental.pallas import tpu as pltpu
from jax.experimental.pallas import tpu_sc as plsc

N = 10000
D = 128
DL = 16
E = 320000
NC = 2
NS = 16
NW = NC * NS
EPW = E // NW
B = 125
NB = EPW // B
RPT = N // NS


def _make_agg_body(with_deg):
    def body(tab_hbm, src_hbm, dst_hbm, zz_hbm, *rest):
        if with_deg:
            (zd_hbm, ones_hbm, agg_hbm, deg_hbm,
             sb0, db0, sb1, db1, rows0, rows1, onesb, acc, degacc,
             semi0, semi1, semg0, semg1) = rest
        else:
            (agg_hbm,
             sb0, db0, sb1, db1, rows0, rows1, acc,
             semi0, semi1, semg0, semg1) = rest
        c = lax.axis_index("c")
        s = lax.axis_index("s")
        wid = s * NC + c
        pltpu.sync_copy(zz_hbm, acc.at[pl.ds(s * RPT, RPT)])
        if with_deg:
            pltpu.sync_copy(zd_hbm, degacc.at[pl.ds(s * RPT, RPT)])
            pltpu.sync_copy(ones_hbm, onesb)
        plsc.subcore_barrier()

        ibase = wid * NB

        def iload(i, sb, db, sem):
            pltpu.async_copy(src_hbm.at[ibase + i], sb, sem)
            pltpu.async_copy(dst_hbm.at[ibase + i], db, sem)

        def iwait(sb, db, sem):
            pltpu.make_async_copy(src_hbm.at[0], sb, sem).wait()
            pltpu.make_async_copy(dst_hbm.at[0], db, sem).wait()

        def gather(sb, rows, sem):
            pltpu.async_copy(tab_hbm.at[sb], rows, sem)

        def gwait(rows, sem):
            pltpu.make_async_copy(tab_hbm.at[sb0], rows, sem).wait()

        def scat(db, rows):
            pltpu.sync_copy(rows, acc.at[db], add=True)
            if with_deg:
                pltpu.sync_copy(onesb, degacc.at[db], add=True)

        iload(0, sb0, db0, semi0)
        iload(1, sb1, db1, semi1)
        iwait(sb0, db0, semi0)
        gather(sb0, rows0, semg0)
        iwait(sb1, db1, semi1)
        gather(sb1, rows1, semg1)

        def step(j, carry):
            i0 = 2 * j
            gwait(rows0, semg0)
            scat(db0, rows0)
            iload(i0 + 2, sb0, db0, semi0)
            gwait(rows1, semg1)
            scat(db1, rows1)
            iload(i0 + 3, sb1, db1, semi1)
            iwait(sb0, db0, semi0)
            gather(sb0, rows0, semg0)
            iwait(sb1, db1, semi1)
            gather(sb1, rows1, semg1)
            return carry

        lax.fori_loop(0, NB // 2 - 1, step, 0)
        gwait(rows0, semg0)
        scat(db0, rows0)
        gwait(rows1, semg1)
        scat(db1, rows1)
        plsc.subcore_barrier()
        pltpu.sync_copy(acc.at[pl.ds(s * RPT, RPT)], agg_hbm.at[c * NS + s])
        if with_deg:
            pltpu.sync_copy(degacc.at[pl.ds(s * RPT, RPT)],
                            deg_hbm.at[c * NS + s])

    return body


_SC_MESH = dict(
    mesh=plsc.VectorSubcoreMesh(core_axis_name="c", subcore_axis_name="s"),
    compiler_params=pltpu.CompilerParams(use_tc_tiling_on_sc=False),
)

_IDX_SCRATCH = (
    pltpu.VMEM((B,), jnp.int32),
    pltpu.VMEM((B,), jnp.int32),
    pltpu.VMEM((B,), jnp.int32),
    pltpu.VMEM((B,), jnp.int32),
    pltpu.VMEM((B, D), jnp.float32),
    pltpu.VMEM((B, D), jnp.float32),
)

_SEMS = (pltpu.SemaphoreType.DMA,) * 4


def _sc_agg_deg(table, src, dst, zz, zd, ones_in):
    return pl.kernel(
        _make_agg_body(True),
        out_type=(jax.ShapeDtypeStruct((NW, RPT, D), jnp.float32),
                  jax.ShapeDtypeStruct((NW, RPT, DL), jnp.float32)),
        scratch_types=_IDX_SCRATCH + (
            pltpu.VMEM((B, DL), jnp.float32),
            pltpu.VMEM_SHARED((N, D), jnp.float32),
            pltpu.VMEM_SHARED((N, DL), jnp.float32),
        ) + _SEMS,
        **_SC_MESH,
    )(table, src, dst, zz, zd, ones_in)


def _sc_agg(table, src, dst, zz):
    return pl.kernel(
        _make_agg_body(False),
        out_type=jax.ShapeDtypeStruct((NW, RPT, D), jnp.float32),
        scratch_types=_IDX_SCRATCH + (
            pltpu.VMEM_SHARED((N, D), jnp.float32),
        ) + _SEMS,
        **_SC_MESH,
    )(table, src, dst, zz)


BLK = 1000
NBLK = N // BLK


def _layer_body(relu, x_ref, a0_ref, a1_ref, d0_ref, d1_ref, ws_ref, wn_ref,
                b_ref, o_ref):
    d = d0_ref[:, 0:1] + d1_ref[:, 0:1]
    rdeg = 1.0 / jnp.maximum(d, 1.0)
    neigh = (a0_ref[...] + a1_ref[...]) * rdeg
    h = (jnp.dot(x_ref[...], ws_ref[...], preferred_element_type=jnp.float32)
         + jnp.dot(neigh, wn_ref[...], preferred_element_type=jnp.float32)
         + b_ref[...])
    o_ref[...] = jnp.maximum(h, 0.0) if relu else h


def _tc_layer(x, agg, degp, ws_t, wn_t, b, relu):
    return pl.pallas_call(
        functools.partial(_layer_body, relu),
        grid=(NBLK,),
        in_specs=[
            pl.BlockSpec((BLK, D), lambda i: (i, 0)),
            pl.BlockSpec((BLK, D), lambda i: (i, 0)),
            pl.BlockSpec((BLK, D), lambda i: (i + NBLK, 0)),
            pl.BlockSpec((BLK, DL), lambda i: (i, 0)),
            pl.BlockSpec((BLK, DL), lambda i: (i + NBLK, 0)),
            pl.BlockSpec((D, D), lambda i: (0, 0)),
            pl.BlockSpec((D, D), lambda i: (0, 0)),
            pl.BlockSpec((1, D), lambda i: (0, 0)),
        ],
        out_specs=pl.BlockSpec((BLK, D), lambda i: (i, 0)),
        out_shape=jax.ShapeDtypeStruct((N, D), jnp.float32),
    )(x, agg, agg, degp, degp, ws_t, wn_t, b)


def kernel(x, edge_index, W_self1, W_neigh1, b1, W_self2, W_neigh2, b2):
    src = edge_index[0].astype(jnp.int32).reshape(NW * NB, B)
    dst = edge_index[1].astype(jnp.int32).reshape(NW * NB, B)
    zz = jnp.zeros((RPT, D), jnp.float32)
    zd = jnp.zeros((RPT, DL), jnp.float32)
    ones_in = jnp.ones((B, DL), jnp.float32)
    agg1, degp = _sc_agg_deg(x, src, dst, zz, zd, ones_in)
    agg1 = agg1.reshape(NC * N, D)
    degp = degp.reshape(NC * N, DL)
    h1 = _tc_layer(x, agg1, degp, W_self1.T, W_neigh1.T, b1[None, :], True)
    agg2 = _sc_agg(h1, src, dst, zz).reshape(NC * N, D)
    return _tc_layer(h1, agg2, degp, W_self2.T, W_neigh2.T, b2[None, :], False)


# speedup vs baseline: 10.1140x; 1.0350x over previous
import functools

import jax
import jax.numpy as jnp
from jax import lax
from jax.experimental import pallas as pl
from jax.experimental.pallas import tpu as pltpu
from jax.experimental.pallas import tpu_sc as plsc

N = 10000
D = 128
DL = 16
E = 320000
NC = 2
NS = 16
NW = NC * NS
EPW = E // NW
B = 125
NB = EPW // B
RPT = N // NS


def _make_agg_body(with_deg):
    def body(tab_hbm, src_hbm, dst_hbm, zz_hbm, *rest):
        if with_deg:
            (zd_hbm, ones_hbm, agg_hbm, deg_hbm,
             sb0, db0, sb1, db1, rows0, rows1, onesb, acc, degacc,
             semi0, semi1, semg0, semg1, sems0, sems1) = rest
        else:
            (agg_hbm,
             sb0, db0, sb1, db1, rows0, rows1, acc,
             semi0, semi1, semg0, semg1, sems0, sems1) = rest
        c = lax.axis_index("c")
        s = lax.axis_index("s")
        wid = s * NC + c
        pltpu.sync_copy(zz_hbm, acc.at[pl.ds(s * RPT, RPT)])
        if with_deg:
            pltpu.sync_copy(zd_hbm, degacc.at[pl.ds(s * RPT, RPT)])
            pltpu.sync_copy(ones_hbm, onesb)
        plsc.subcore_barrier()

        ibase = wid * NB

        def iload(i, sb, db, sem):
            pltpu.async_copy(src_hbm.at[ibase + i], sb, sem)
            pltpu.async_copy(dst_hbm.at[ibase + i], db, sem)

        def iwait(sb, db, sem):
            pltpu.make_async_copy(src_hbm.at[0], sb, sem).wait()
            pltpu.make_async_copy(dst_hbm.at[0], db, sem).wait()

        def gather(sb, rows, sem):
            pltpu.async_copy(tab_hbm.at[sb], rows, sem)

        def gwait(rows, sem):
            pltpu.make_async_copy(tab_hbm.at[sb0], rows, sem).wait()

        def scat(db, rows, sem):
            pltpu.async_copy(rows, acc.at[db], sem, add=True)
            if with_deg:
                pltpu.async_copy(onesb, degacc.at[db], sem, add=True)

        def swait(db, rows, sem):
            pltpu.make_async_copy(rows, acc.at[db], sem).wait()
            if with_deg:
                pltpu.make_async_copy(onesb, degacc.at[db], sem).wait()

        iload(0, sb0, db0, semi0)
        iload(1, sb1, db1, semi1)
        iwait(sb0, db0, semi0)
        gather(sb0, rows0, semg0)
        iwait(sb1, db1, semi1)
        gather(sb1, rows1, semg1)

        def step(j, carry):
            i0 = 2 * j
            gwait(rows0, semg0)
            scat(db0, rows0, sems0)
            gwait(rows1, semg1)
            scat(db1, rows1, sems1)
            swait(db0, rows0, sems0)
            iload(i0 + 2, sb0, db0, semi0)
            iwait(sb0, db0, semi0)
            gather(sb0, rows0, semg0)
            swait(db1, rows1, sems1)
            iload(i0 + 3, sb1, db1, semi1)
            iwait(sb1, db1, semi1)
            gather(sb1, rows1, semg1)
            return carry

        lax.fori_loop(0, NB // 2 - 1, step, 0)
        gwait(rows0, semg0)
        scat(db0, rows0, sems0)
        gwait(rows1, semg1)
        scat(db1, rows1, sems1)
        swait(db0, rows0, sems0)
        swait(db1, rows1, sems1)
        plsc.subcore_barrier()
        pltpu.sync_copy(acc.at[pl.ds(s * RPT, RPT)], agg_hbm.at[c * NS + s])
        if with_deg:
            pltpu.sync_copy(degacc.at[pl.ds(s * RPT, RPT)],
                            deg_hbm.at[c * NS + s])

    return body


_SC_MESH = dict(
    mesh=plsc.VectorSubcoreMesh(core_axis_name="c", subcore_axis_name="s"),
    compiler_params=pltpu.CompilerParams(use_tc_tiling_on_sc=False),
)

_IDX_SCRATCH = (
    pltpu.VMEM((B,), jnp.int32),
    pltpu.VMEM((B,), jnp.int32),
    pltpu.VMEM((B,), jnp.int32),
    pltpu.VMEM((B,), jnp.int32),
    pltpu.VMEM((B, D), jnp.float32),
    pltpu.VMEM((B, D), jnp.float32),
)

_SEMS = (pltpu.SemaphoreType.DMA,) * 6


def _sc_agg_deg(table, src, dst, zz, zd, ones_in):
    return pl.kernel(
        _make_agg_body(True),
        out_type=(jax.ShapeDtypeStruct((NW, RPT, D), jnp.float32),
                  jax.ShapeDtypeStruct((NW, RPT, DL), jnp.float32)),
        scratch_types=_IDX_SCRATCH + (
            pltpu.VMEM((B, DL), jnp.float32),
            pltpu.VMEM_SHARED((N, D), jnp.float32),
            pltpu.VMEM_SHARED((N, DL), jnp.float32),
        ) + _SEMS,
        **_SC_MESH,
    )(table, src, dst, zz, zd, ones_in)


def _sc_agg(table, src, dst, zz):
    return pl.kernel(
        _make_agg_body(False),
        out_type=jax.ShapeDtypeStruct((NW, RPT, D), jnp.float32),
        scratch_types=_IDX_SCRATCH + (
            pltpu.VMEM_SHARED((N, D), jnp.float32),
        ) + _SEMS,
        **_SC_MESH,
    )(table, src, dst, zz)


BLK = 1000
NBLK = N // BLK


def _layer_body(relu, x_ref, a0_ref, a1_ref, d0_ref, d1_ref, ws_ref, wn_ref,
                b_ref, o_ref):
    d = d0_ref[:, 0:1] + d1_ref[:, 0:1]
    rdeg = 1.0 / jnp.maximum(d, 1.0)
    neigh = (a0_ref[...] + a1_ref[...]) * rdeg
    h = (jnp.dot(x_ref[...], ws_ref[...], preferred_element_type=jnp.float32)
         + jnp.dot(neigh, wn_ref[...], preferred_element_type=jnp.float32)
         + b_ref[...])
    o_ref[...] = jnp.maximum(h, 0.0) if relu else h


def _tc_layer(x, agg, degp, ws_t, wn_t, b, relu):
    return pl.pallas_call(
        functools.partial(_layer_body, relu),
        grid=(NBLK,),
        in_specs=[
            pl.BlockSpec((BLK, D), lambda i: (i, 0)),
            pl.BlockSpec((BLK, D), lambda i: (i, 0)),
            pl.BlockSpec((BLK, D), lambda i: (i + NBLK, 0)),
            pl.BlockSpec((BLK, DL), lambda i: (i, 0)),
            pl.BlockSpec((BLK, DL), lambda i: (i + NBLK, 0)),
            pl.BlockSpec((D, D), lambda i: (0, 0)),
            pl.BlockSpec((D, D), lambda i: (0, 0)),
            pl.BlockSpec((1, D), lambda i: (0, 0)),
        ],
        out_specs=pl.BlockSpec((BLK, D), lambda i: (i, 0)),
        out_shape=jax.ShapeDtypeStruct((N, D), jnp.float32),
    )(x, agg, agg, degp, degp, ws_t, wn_t, b)


def kernel(x, edge_index, W_self1, W_neigh1, b1, W_self2, W_neigh2, b2):
    src = edge_index[0].astype(jnp.int32).reshape(NW * NB, B)
    dst = edge_index[1].astype(jnp.int32).reshape(NW * NB, B)
    zz = jnp.zeros((RPT, D), jnp.float32)
    zd = jnp.zeros((RPT, DL), jnp.float32)
    ones_in = jnp.ones((B, DL), jnp.float32)
    agg1, degp = _sc_agg_deg(x, src, dst, zz, zd, ones_in)
    agg1 = agg1.reshape(NC * N, D)
    degp = degp.reshape(NC * N, DL)
    h1 = _tc_layer(x, agg1, degp, W_self1.T, W_neigh1.T, b1[None, :], True)
    agg2 = _sc_agg(h1, src, dst, zz).reshape(NC * N, D)
    return _tc_layer(h1, agg2, degp, W_self2.T, W_neigh2.T, b2[None, :], False)


# speedup vs baseline: 11.9757x; 1.1841x over previous
import functools

import jax
import jax.numpy as jnp
from jax import lax
from jax.experimental import pallas as pl
from jax.experimental.pallas import tpu as pltpu
from jax.experimental.pallas import tpu_sc as plsc

N = 10000
D = 128
DL = 16
E = 320000
NC = 2
NS = 16
NW = NC * NS
EPW = E // NW
B = 125
NB = EPW // B
RPT = N // NS


def _make_agg_body(with_deg):
    def body(tab_hbm, src_hbm, dst_hbm, zz_hbm, *rest):
        if with_deg:
            (zd_hbm, ones_hbm, agg_hbm, deg_hbm,
             sb0, db0, sb1, db1, sb2, db2, sb3, db3,
             rows0, rows1, onesb, acc, degacc,
             semi0, semi1, semi2, semi3, semg0, semg1, sems0, sems1) = rest
        else:
            (agg_hbm,
             sb0, db0, sb1, db1, sb2, db2, sb3, db3,
             rows0, rows1, acc,
             semi0, semi1, semi2, semi3, semg0, semg1, sems0, sems1) = rest
        c = lax.axis_index("c")
        s = lax.axis_index("s")
        wid = s * NC + c
        pltpu.sync_copy(zz_hbm, acc.at[pl.ds(s * RPT, RPT)])
        if with_deg:
            pltpu.sync_copy(zd_hbm, degacc.at[pl.ds(s * RPT, RPT)])
            pltpu.sync_copy(ones_hbm, onesb)
        plsc.subcore_barrier()

        ibase = wid * NB

        def iload(i, sb, db, sem):
            pltpu.async_copy(src_hbm.at[ibase + i], sb, sem)
            pltpu.async_copy(dst_hbm.at[ibase + i], db, sem)

        def iwait(sb, db, sem):
            pltpu.make_async_copy(src_hbm.at[0], sb, sem).wait()
            pltpu.make_async_copy(dst_hbm.at[0], db, sem).wait()

        def gather(sb, rows, sem):
            pltpu.async_copy(tab_hbm.at[sb], rows, sem)

        def gwait(rows, sem):
            pltpu.make_async_copy(tab_hbm.at[sb0], rows, sem).wait()

        def scat(db, rows, sem):
            pltpu.async_copy(rows, acc.at[db], sem, add=True)
            if with_deg:
                pltpu.async_copy(onesb, degacc.at[db], sem, add=True)

        def swait(db, rows, sem):
            pltpu.make_async_copy(rows, acc.at[db], sem).wait()
            if with_deg:
                pltpu.make_async_copy(onesb, degacc.at[db], sem).wait()

        iload(0, sb0, db0, semi0)
        iload(1, sb1, db1, semi1)
        iload(2, sb2, db2, semi2)
        iload(3, sb3, db3, semi3)
        iwait(sb0, db0, semi0)
        gather(sb0, rows0, semg0)
        iwait(sb1, db1, semi1)
        gather(sb1, rows1, semg1)

        def quarter(i_next, sb_c, db_c, semi_c, sb_n, db_n, semi_n,
                    rows, semg, sems):
            gwait(rows, semg)
            scat(db_c, rows, sems)
            swait(db_c, rows, sems)
            iload(i_next, sb_c, db_c, semi_c)
            iwait(sb_n, db_n, semi_n)
            gather(sb_n, rows, semg)

        def step(j, carry):
            i0 = 4 * j
            quarter(i0 + 4, sb0, db0, semi0, sb2, db2, semi2,
                    rows0, semg0, sems0)
            quarter(i0 + 5, sb1, db1, semi1, sb3, db3, semi3,
                    rows1, semg1, sems1)
            quarter(i0 + 6, sb2, db2, semi2, sb0, db0, semi0,
                    rows0, semg0, sems0)
            quarter(i0 + 7, sb3, db3, semi3, sb1, db1, semi1,
                    rows1, semg1, sems1)
            return carry

        lax.fori_loop(0, NB // 4 - 1, step, 0)
        gwait(rows0, semg0)
        scat(db0, rows0, sems0)
        swait(db0, rows0, sems0)
        iwait(sb2, db2, semi2)
        gather(sb2, rows0, semg0)
        gwait(rows1, semg1)
        scat(db1, rows1, sems1)
        swait(db1, rows1, sems1)
        iwait(sb3, db3, semi3)
        gather(sb3, rows1, semg1)
        gwait(rows0, semg0)
        scat(db2, rows0, sems0)
        swait(db2, rows0, sems0)
        gwait(rows1, semg1)
        scat(db3, rows1, sems1)
        swait(db3, rows1, sems1)
        plsc.subcore_barrier()
        pltpu.sync_copy(acc.at[pl.ds(s * RPT, RPT)], agg_hbm.at[c * NS + s])
        if with_deg:
            pltpu.sync_copy(degacc.at[pl.ds(s * RPT, RPT)],
                            deg_hbm.at[c * NS + s])

    return body


_SC_MESH = dict(
    mesh=plsc.VectorSubcoreMesh(core_axis_name="c", subcore_axis_name="s"),
    compiler_params=pltpu.CompilerParams(use_tc_tiling_on_sc=False),
)

_IDX_SCRATCH = (
    (pltpu.VMEM((B,), jnp.int32),) * 8
    + (
        pltpu.VMEM((B, D), jnp.float32),
        pltpu.VMEM((B, D), jnp.float32),
    )
)

_SEMS = (pltpu.SemaphoreType.DMA,) * 8


def _sc_agg_deg(table, src, dst, zz, zd, ones_in):
    return pl.kernel(
        _make_agg_body(True),
        out_type=(jax.ShapeDtypeStruct((NW, RPT, D), jnp.float32),
                  jax.ShapeDtypeStruct((NW, RPT, DL), jnp.float32)),
        scratch_types=_IDX_SCRATCH + (
            pltpu.VMEM((B, DL), jnp.float32),
            pltpu.VMEM_SHARED((N, D), jnp.float32),
            pltpu.VMEM_SHARED((N, DL), jnp.float32),
        ) + _SEMS,
        **_SC_MESH,
    )(table, src, dst, zz, zd, ones_in)


def _sc_agg(table, src, dst, zz):
    return pl.kernel(
        _make_agg_body(False),
        out_type=jax.ShapeDtypeStruct((NW, RPT, D), jnp.float32),
        scratch_types=_IDX_SCRATCH + (
            pltpu.VMEM_SHARED((N, D), jnp.float32),
        ) + _SEMS,
        **_SC_MESH,
    )(table, src, dst, zz)


BLK = 1000
NBLK = N // BLK


def _layer_body(relu, x_ref, a0_ref, a1_ref, d0_ref, d1_ref, ws_ref, wn_ref,
                b_ref, o_ref):
    d = d0_ref[:, 0:1] + d1_ref[:, 0:1]
    rdeg = 1.0 / jnp.maximum(d, 1.0)
    neigh = (a0_ref[...] + a1_ref[...]) * rdeg
    h = (jnp.dot(x_ref[...], ws_ref[...], preferred_element_type=jnp.float32)
         + jnp.dot(neigh, wn_ref[...], preferred_element_type=jnp.float32)
         + b_ref[...])
    o_ref[...] = jnp.maximum(h, 0.0) if relu else h


def _tc_layer(x, agg, degp, ws_t, wn_t, b, relu):
    return pl.pallas_call(
        functools.partial(_layer_body, relu),
        grid=(NBLK,),
        in_specs=[
            pl.BlockSpec((BLK, D), lambda i: (i, 0)),
            pl.BlockSpec((BLK, D), lambda i: (i, 0)),
            pl.BlockSpec((BLK, D), lambda i: (i + NBLK, 0)),
            pl.BlockSpec((BLK, DL), lambda i: (i, 0)),
            pl.BlockSpec((BLK, DL), lambda i: (i + NBLK, 0)),
            pl.BlockSpec((D, D), lambda i: (0, 0)),
            pl.BlockSpec((D, D), lambda i: (0, 0)),
            pl.BlockSpec((1, D), lambda i: (0, 0)),
        ],
        out_specs=pl.BlockSpec((BLK, D), lambda i: (i, 0)),
        out_shape=jax.ShapeDtypeStruct((N, D), jnp.float32),
    )(x, agg, agg, degp, degp, ws_t, wn_t, b)


def kernel(x, edge_index, W_self1, W_neigh1, b1, W_self2, W_neigh2, b2):
    src = edge_index[0].astype(jnp.int32).reshape(NW * NB, B)
    dst = edge_index[1].astype(jnp.int32).reshape(NW * NB, B)
    zz = jnp.zeros((RPT, D), jnp.float32)
    zd = jnp.zeros((RPT, DL), jnp.float32)
    ones_in = jnp.ones((B, DL), jnp.float32)
    agg1, degp = _sc_agg_deg(x, src, dst, zz, zd, ones_in)
    agg1 = agg1.reshape(NC * N, D)
    degp = degp.reshape(NC * N, DL)
    h1 = _tc_layer(x, agg1, degp, W_self1.T, W_neigh1.T, b1[None, :], True)
    agg2 = _sc_agg(h1, src, dst, zz).reshape(NC * N, D)
    return _tc_layer(h1, agg2, degp, W_self2.T, W_neigh2.T, b2[None, :], False)
